# baseline (speedup 1.0000x reference)
import functools

import jax
import jax.numpy as jnp
from jax import lax
from jax.experimental import pallas as pl
from jax.experimental.pallas import tpu as pltpu
from jax.experimental.pallas import tpu_sc as plsc

NC = 2
NS = 16
NW = NC * NS

BATCH = 4096
SEQ = 200
D = 64
NB = BATCH // 128
UNITS = SEQ * NB
UPW = UNITS // NW
NBUF = 4


def _mesh():
    return plsc.VectorSubcoreMesh(core_axis_name="c", subcore_axis_name="s")


@functools.partial(
    pl.kernel,
    out_type=jax.ShapeDtypeStruct((SEQ, D // 8, NB, 8, 128), jnp.float32),
    mesh=_mesh(),
    scratch_types=[
        *[pltpu.VMEM((128,), jnp.int32) for _ in range(NBUF)],
        *[pltpu.VMEM((128, D), jnp.float32) for _ in range(NBUF)],
        *[pltpu.VMEM((D // 8, 8, 128), jnp.float32) for _ in range(NBUF)],
        *[pltpu.SemaphoreType.DMA for _ in range(3 * NBUF)],
    ],
    compiler_params=pltpu.CompilerParams(
        use_tc_tiling_on_sc=False, needs_layout_passes=False
    ),
)
def _embed_lookup(xt_hbm, table_hbm, out_hbm, *bufs_sems):
    idx = bufs_sems[:NBUF]
    rows = bufs_sems[NBUF : 2 * NBUF]
    tbuf = bufs_sems[2 * NBUF : 3 * NBUF]
    isem = bufs_sems[3 * NBUF : 4 * NBUF]
    gsem = bufs_sems[4 * NBUF : 5 * NBUF]
    osem = bufs_sems[5 * NBUF :]
    wid = lax.axis_index("s") * NC + lax.axis_index("c")
    u0 = wid * UPW

    row_ids = [lax.iota(jnp.int32, 16) + lg * 16 for lg in range(8)]

    def start_idx(j, slot):
        u = u0 + j
        pltpu.async_copy(
            xt_hbm.at[u // NB, pl.ds((u % NB) * 128, 128)], idx[slot], isem[slot]
        )

    def wait_idx(j, slot):
        u = u0 + j
        pltpu.make_async_copy(
            xt_hbm.at[u // NB, pl.ds((u % NB) * 128, 128)], idx[slot], isem[slot]
        ).wait()

    def start_gather(slot):
        pltpu.async_copy(table_hbm.at[idx[slot]], rows[slot], gsem[slot])

    def wait_gather(slot):
        pltpu.make_async_copy(
            table_hbm.at[idx[slot]], rows[slot], gsem[slot]
        ).wait()

    def start_out(j, slot):
        u = u0 + j
        pltpu.async_copy(tbuf[slot], out_hbm.at[u // NB, :, u % NB], osem[slot])

    def wait_out(j, slot):
        u = u0 + j
        pltpu.make_async_copy(
            tbuf[slot], out_hbm.at[u // NB, :, u % NB], osem[slot]
        ).wait()

    def transpose_into(slot):
        for Db in range(D // 8):
            for u in range(8):
                col = jnp.full((16,), Db * 8 + u, jnp.int32)
                for lg in range(8):
                    vals = plsc.load_gather(rows[slot], [row_ids[lg], col])
                    tbuf[slot][Db, u, pl.ds(lg * 16, 16)] = vals

    for slot in range(NBUF):
        start_idx(slot, slot)
    for slot in range(NBUF):
        wait_idx(slot, slot)
        start_gather(slot)

    @pl.loop(0, UPW, step=NBUF)
    def _(j0):
        for slot in range(NBUF):
            j = j0 + slot
            wait_gather(slot)

            @pl.when(j + NBUF < UPW)
            def _():
                start_idx(j + NBUF, slot)

            @pl.when(j >= NBUF)
            def _():
                wait_out(j - NBUF, slot)

            transpose_into(slot)
            start_out(j, slot)

            @pl.when(j + NBUF < UPW)
            def _():
                wait_idx(j + NBUF, slot)
                start_gather(slot)

    for slot in range(NBUF):
        wait_out(UPW - NBUF + slot, slot)


def kernel(x, embed_word):
    out_v = _embed_lookup(x.T, embed_word)
    return out_v.transpose(2, 4, 0, 1, 3).reshape(BATCH, SEQ, D)


# speedup vs baseline: 1.0030x; 1.0030x over previous
import functools

import jax
import jax.numpy as jnp
from jax import lax
from jax.experimental import pallas as pl
from jax.experimental.pallas import tpu as pltpu
from jax.experimental.pallas import tpu_sc as plsc

NC = 2
NS = 16
NW = NC * NS

BATCH = 4096
SEQ = 200
D = 64
NB = BATCH // 128
UNITS = SEQ * NB
UPW = UNITS // NW
NBUF = 4


def _mesh():
    return plsc.VectorSubcoreMesh(core_axis_name="c", subcore_axis_name="s")


@functools.partial(
    pl.kernel,
    out_type=jax.ShapeDtypeStruct((SEQ, D // 8, NB, 8, 128), jnp.float32),
    mesh=_mesh(),
    scratch_types=[
        *[pltpu.VMEM((128,), jnp.int32) for _ in range(NBUF)],
        *[pltpu.VMEM((128, D), jnp.float32) for _ in range(NBUF)],
        *[pltpu.VMEM((D // 8, 8, 128), jnp.float32) for _ in range(NBUF)],
        *[pltpu.SemaphoreType.DMA for _ in range(3 * NBUF)],
    ],
    compiler_params=pltpu.CompilerParams(
        use_tc_tiling_on_sc=False,
        needs_layout_passes=False,
        disable_bounds_checks=True,
    ),
)
def _embed_lookup(xt_hbm, table_hbm, out_hbm, *bufs_sems):
    idx = bufs_sems[:NBUF]
    rows = bufs_sems[NBUF : 2 * NBUF]
    tbuf = bufs_sems[2 * NBUF : 3 * NBUF]
    isem = bufs_sems[3 * NBUF : 4 * NBUF]
    gsem = bufs_sems[4 * NBUF : 5 * NBUF]
    osem = bufs_sems[5 * NBUF :]
    wid = lax.axis_index("s") * NC + lax.axis_index("c")
    u0 = wid * UPW

    row_ids = [lax.iota(jnp.int32, 16) + lg * 16 for lg in range(8)]

    def start_idx(j, slot):
        u = u0 + j
        pltpu.async_copy(
            xt_hbm.at[u // NB, pl.ds((u % NB) * 128, 128)], idx[slot], isem[slot]
        )

    def wait_idx(j, slot):
        u = u0 + j
        pltpu.make_async_copy(
            xt_hbm.at[u // NB, pl.ds((u % NB) * 128, 128)], idx[slot], isem[slot]
        ).wait()

    def start_gather(slot):
        pltpu.async_copy(table_hbm.at[idx[slot]], rows[slot], gsem[slot])

    def wait_gather(slot):
        pltpu.make_async_copy(
            table_hbm.at[idx[slot]], rows[slot], gsem[slot]
        ).wait()

    def start_out(j, slot):
        u = u0 + j
        pltpu.async_copy(tbuf[slot], out_hbm.at[u // NB, :, u % NB], osem[slot])

    def wait_out(j, slot):
        u = u0 + j
        pltpu.make_async_copy(
            tbuf[slot], out_hbm.at[u // NB, :, u % NB], osem[slot]
        ).wait()

    def transpose_into(slot):
        for Db in range(D // 8):
            for u in range(8):
                col = jnp.full((16,), Db * 8 + u, jnp.int32)
                for lg in range(8):
                    vals = plsc.load_gather(rows[slot], [row_ids[lg], col])
                    tbuf[slot][Db, u, pl.ds(lg * 16, 16)] = vals

    for slot in range(NBUF):
        start_idx(slot, slot)
    for slot in range(NBUF):
        wait_idx(slot, slot)
        start_gather(slot)

    @pl.loop(0, UPW, step=NBUF)
    def _(j0):
        for slot in range(NBUF):
            j = j0 + slot
            wait_gather(slot)

            @pl.when(j + NBUF < UPW)
            def _():
                start_idx(j + NBUF, slot)

            @pl.when(j >= NBUF)
            def _():
                wait_out(j - NBUF, slot)

            transpose_into(slot)
            start_out(j, slot)

            @pl.when(j + NBUF < UPW)
            def _():
                wait_idx(j + NBUF, slot)
                start_gather(slot)

    for slot in range(NBUF):
        wait_out(UPW - NBUF + slot, slot)


def kernel(x, embed_word):
    out_v = _embed_lookup(x.T, embed_word)
    return out_v.transpose(2, 4, 0, 1, 3).reshape(BATCH, SEQ, D)


# speedup vs baseline: 1.6895x; 1.6845x over previous
import functools

import jax
import jax.numpy as jnp
from jax import lax
from jax.experimental import pallas as pl
from jax.experimental.pallas import tpu as pltpu
from jax.experimental.pallas import tpu_sc as plsc

NC = 2
NS = 16
NW = NC * NS

BATCH = 4096
SEQ = 200
D = 64
BPW = BATCH // NW
CA, CB = 128, 72
NBUF = 4


def _mesh():
    return plsc.VectorSubcoreMesh(core_axis_name="c", subcore_axis_name="s")


@functools.partial(
    pl.kernel,
    out_type=jax.ShapeDtypeStruct((BATCH, SEQ, D), jnp.float32),
    mesh=_mesh(),
    scratch_types=[
        pltpu.VMEM((BPW, SEQ), jnp.int32),
        *[pltpu.VMEM((SEQ, D), jnp.float32) for _ in range(NBUF)],
        *[pltpu.SemaphoreType.DMA for _ in range(2 * NBUF)],
    ],
    compiler_params=pltpu.CompilerParams(use_tc_tiling_on_sc=False),
)
def _embed_lookup(x_hbm, table_hbm, out_hbm, idx_v, *bufs_sems):
    rows = bufs_sems[:NBUF]
    gsem = bufs_sems[NBUF : 2 * NBUF]
    osem = bufs_sems[2 * NBUF :]
    wid = lax.axis_index("s") * NC + lax.axis_index("c")
    b0 = wid * BPW

    pltpu.sync_copy(x_hbm.at[pl.ds(b0, BPW)], idx_v)

    def start_gathers(b, slot):
        pltpu.async_copy(
            table_hbm.at[idx_v.at[b, pl.ds(0, CA)]],
            rows[slot].at[pl.ds(0, CA)],
            gsem[slot],
        )
        pltpu.async_copy(
            table_hbm.at[idx_v.at[b, pl.ds(CA, CB)]],
            rows[slot].at[pl.ds(CA, CB)],
            gsem[slot],
        )

    def wait_gathers(b, slot):
        pltpu.make_async_copy(
            table_hbm.at[idx_v.at[b, pl.ds(0, CA)]],
            rows[slot].at[pl.ds(0, CA)],
            gsem[slot],
        ).wait()
        pltpu.make_async_copy(
            table_hbm.at[idx_v.at[b, pl.ds(CA, CB)]],
            rows[slot].at[pl.ds(CA, CB)],
            gsem[slot],
        ).wait()

    for slot in range(NBUF):
        start_gathers(slot, slot)

    @pl.loop(0, BPW, step=NBUF)
    def _(j):
        for slot in range(NBUF):
            b = j + slot
            wait_gathers(b, slot)
            pltpu.async_copy(rows[slot], out_hbm.at[b0 + b], osem[slot])
        for slot in range(NBUF):
            b = j + slot
            pltpu.make_async_copy(
                rows[slot], out_hbm.at[b0 + b], osem[slot]
            ).wait()

            @pl.when(b + NBUF < BPW)
            def _():
                start_gathers(b + NBUF, slot)


def kernel(x, embed_word):
    return _embed_lookup(x, embed_word)
